# baseline (speedup 1.0000x reference)
import jax
import jax.numpy as jnp
from jax.experimental import pallas as pl

_SLOPE = (1.0 / 8.0 + 1.0 / 3.0) / 2.0


def _body(x_ref, w0_ref, w1_ref, b_ref, o_ref):
    x = x_ref[...]
    neg = x < 0
    h0 = jnp.where(neg, x * _SLOPE, x)
    h1 = jnp.where(neg, x * (_SLOPE * _SLOPE), x)
    acc = jnp.dot(h0, w0_ref[...], preferred_element_type=jnp.float32)
    acc = acc + jnp.dot(h1, w1_ref[...], preferred_element_type=jnp.float32)
    o_ref[...] = acc + b_ref[...]


def kernel(node_feats, edge_feats, edge_index, Wn0, bn0, Wl0, bl0, Wn1, bn1, Wl1, bl1, Wo, bo):
    n, d = node_feats.shape
    h = Wo.shape[1]
    tile = 2000
    w0 = Wo[:d]
    w1 = Wo[d:]
    b = bo.reshape(1, h)
    return pl.pallas_call(
        _body,
        grid=(n // tile,),
        in_specs=[
            pl.BlockSpec((tile, d), lambda i: (i, 0)),
            pl.BlockSpec((d, h), lambda i: (0, 0)),
            pl.BlockSpec((d, h), lambda i: (0, 0)),
            pl.BlockSpec((1, h), lambda i: (0, 0)),
        ],
        out_specs=pl.BlockSpec((tile, h), lambda i: (i, 0)),
        out_shape=jax.ShapeDtypeStruct((n, h), jnp.float32),
    )(node_feats, w0, w1, b)
